# baseline (speedup 1.0000x reference)
import functools

import jax
import jax.numpy as jnp
from jax import lax
from jax.experimental import pallas as pl
from jax.experimental.pallas import tpu as pltpu
from jax.experimental.pallas import tpu_sc as plsc

CHUNK = 128
RING = 8
AHEAD = 4
BR = 2048
TB = 32


def _pack_kernel(p_ref, q_ref, o_ref):
    o_ref[...] = jnp.concatenate([p_ref[...].T, q_ref[...].T], axis=1)


def _pack_table(table):
    V, D = table.shape
    H = V // 2
    tT = table.T
    grid = (H + BR - 1) // BR
    return pl.pallas_call(
        _pack_kernel,
        grid=(grid,),
        in_specs=[
            pl.BlockSpec((D, BR), lambda i: (0, 2 * i)),
            pl.BlockSpec(
                (D, BR), lambda i: (0, jnp.minimum(2 * i + 1, V // BR))
            ),
        ],
        out_specs=pl.BlockSpec((BR, 2 * D), lambda i: (i, 0)),
        out_shape=jax.ShapeDtypeStruct((grid * BR, 2 * D), jnp.float32),
    )(tT, tT)


def _chunk_transpose_kernel(i_ref, o_ref):
    BT = i_ref.shape[0] // 64
    D = i_ref.shape[1] // 2
    for bt in range(BT):
        pr = i_ref[pl.ds(bt * 64, 64)]
        t = jnp.concatenate([pr[:, :D].T, pr[:, D:].T], axis=1)
        for dt in range(D // 8):
            o_ref[dt * BT + bt] = t[dt * 8:(dt + 1) * 8]


def _chunk_transpose(lin128, F, BT, D):
    DT = D // 8
    return pl.pallas_call(
        _chunk_transpose_kernel,
        grid=(F,),
        in_specs=[pl.BlockSpec((BT * 64, 2 * D), lambda i: (i, 0))],
        out_specs=pl.BlockSpec((DT * BT, 8, CHUNK), lambda i: (i, 0, 0)),
        out_shape=jax.ShapeDtypeStruct((F * DT * BT, 8, CHUNK), jnp.float32),
    )(lin128)


def kernel(x, table):
    B, F = x.shape
    V, D = table.shape
    N = B * F
    BT = B // CHUNK
    n_chunks = F * BT
    assert N == n_chunks * CHUNK

    info = plsc.get_sparse_core_info()
    NC, NS = info.num_cores, info.num_subcores
    NW = NC * NS
    assert n_chunks % NW == 0
    cpw = n_chunks // NW
    assert cpw % RING == 0

    xi = x.astype(jnp.int32)
    u = xi % (2 * BR)
    base = xi - u
    v2 = jnp.where(u < BR, 2 * u, 2 * (u - BR) + 1)
    idx2d = (base + v2).T.reshape(n_chunks, 2, CHUNK // 2)
    idx2d = idx2d.transpose(0, 2, 1).reshape(n_chunks, CHUNK)
    tlin = _pack_table(table)
    tlin = tlin.reshape(tlin.shape[0] * 2, D)
    mesh = plsc.VectorSubcoreMesh(core_axis_name="c", subcore_axis_name="s")

    @functools.partial(
        pl.kernel,
        mesh=mesh,
        compiler_params=pltpu.CompilerParams(use_tc_tiling_on_sc=False),
        out_type=jax.ShapeDtypeStruct((N, D), jnp.float32),
        scratch_types=[
            pltpu.VMEM((cpw, CHUNK), jnp.int32),
        ]
        + [pltpu.VMEM((CHUNK, D), jnp.float32) for _ in range(RING)]
        + [pltpu.SemaphoreType.DMA for _ in range(2 * RING)],
    )
    def emb(idx_hbm, table_hbm, out_hbm, idx_v, *bufs):
        rows = bufs[:RING]
        gsem = bufs[RING:2 * RING]
        wsem = bufs[2 * RING:3 * RING]
        wid = lax.axis_index("s") * NC + lax.axis_index("c")
        c0 = wid * cpw
        pltpu.sync_copy(idx_hbm.at[pl.ds(c0, cpw)], idx_v)

        for b in range(AHEAD):
            pltpu.async_copy(table_hbm.at[idx_v.at[b]], rows[b], gsem[b])

        def body(i, carry):
            j0 = i * RING
            for b in range(RING):
                j = j0 + b
                jn = j + AHEAD
                bn = (b + AHEAD) % RING

                @pl.when(jnp.logical_and(jn < cpw, jn >= RING))
                def _drain_write():
                    pltpu.make_async_copy(
                        rows[bn], out_hbm.at[pl.ds(0, CHUNK)], wsem[bn]
                    ).wait()

                @pl.when(jn < cpw)
                def _fire_gather():
                    pltpu.async_copy(
                        table_hbm.at[idx_v.at[jn]], rows[bn], gsem[bn]
                    )

                pltpu.make_async_copy(
                    table_hbm.at[idx_v.at[j]], rows[b], gsem[b]
                ).wait()
                pltpu.async_copy(
                    rows[b], out_hbm.at[pl.ds((c0 + j) * CHUNK, CHUNK)], wsem[b]
                )
            return carry

        lax.fori_loop(0, cpw // RING, body, 0)

        for b in range(RING):
            pltpu.make_async_copy(
                rows[b], out_hbm.at[pl.ds(0, CHUNK)], wsem[b]
            ).wait()

    lin = emb(idx2d, tlin)
    lin128 = lin.reshape(N // 2, 2 * D)
    ot = _chunk_transpose(lin128, F, BT, D)
    o5 = ot.reshape(F, D // 8, BT, 8, CHUNK)
    return o5.transpose((2, 4, 0, 1, 3)).reshape(B, F, D)


# speedup vs baseline: 1.3383x; 1.3383x over previous
import functools

import jax
import jax.numpy as jnp
from jax import lax
from jax.experimental import pallas as pl
from jax.experimental.pallas import tpu as pltpu
from jax.experimental.pallas import tpu_sc as plsc

CHUNK = 128
RING = 8
AHEAD = 4
BR = 2048
TB = 32


def _pack_kernel(p_ref, q_ref, o_ref):
    o_ref[...] = jnp.concatenate([p_ref[...].T, q_ref[...].T], axis=1)


def _pack_table(table):
    V, D = table.shape
    H = V // 2
    tT = table.T
    grid = (H + BR - 1) // BR
    return pl.pallas_call(
        _pack_kernel,
        grid=(grid,),
        in_specs=[
            pl.BlockSpec((D, BR), lambda i: (0, 2 * i)),
            pl.BlockSpec(
                (D, BR), lambda i: (0, jnp.minimum(2 * i + 1, V // BR))
            ),
        ],
        out_specs=pl.BlockSpec((BR, 2 * D), lambda i: (i, 0)),
        out_shape=jax.ShapeDtypeStruct((grid * BR, 2 * D), jnp.float32),
    )(tT, tT)


def _chunk_transpose_kernel(i_ref, o_ref):
    BT = i_ref.shape[0] // CHUNK
    DT = i_ref.shape[1] // 8
    for bt in range(BT):
        t = i_ref[pl.ds(bt * CHUNK, CHUNK)].T
        for dt in range(DT):
            o_ref[dt * BT + bt] = t[dt * 8:(dt + 1) * 8]


def _chunk_transpose(lin, F, BT, D):
    DT = D // 8
    return pl.pallas_call(
        _chunk_transpose_kernel,
        grid=(F,),
        in_specs=[pl.BlockSpec((BT * CHUNK, D), lambda i: (i, 0))],
        out_specs=pl.BlockSpec((DT * BT, 8, CHUNK), lambda i: (i, 0, 0)),
        out_shape=jax.ShapeDtypeStruct((F * DT * BT, 8, CHUNK), jnp.float32),
    )(lin)


def kernel(x, table):
    B, F = x.shape
    V, D = table.shape
    N = B * F
    BT = B // CHUNK
    n_chunks = F * BT
    assert N == n_chunks * CHUNK

    info = plsc.get_sparse_core_info()
    NC, NS = info.num_cores, info.num_subcores
    NW = NC * NS
    assert n_chunks % NW == 0
    cpw = n_chunks // NW
    assert cpw % RING == 0

    xi = x.astype(jnp.int32)
    u = xi % (2 * BR)
    base = xi - u
    v2 = jnp.where(u < BR, 2 * u, 2 * (u - BR) + 1)
    idx2d = (base + v2).T.reshape(n_chunks, CHUNK)
    tlin = _pack_table(table)
    tlin = tlin.reshape(tlin.shape[0] * 2, D)
    mesh = plsc.VectorSubcoreMesh(core_axis_name="c", subcore_axis_name="s")

    @functools.partial(
        pl.kernel,
        mesh=mesh,
        compiler_params=pltpu.CompilerParams(use_tc_tiling_on_sc=False),
        out_type=jax.ShapeDtypeStruct((N, D), jnp.float32),
        scratch_types=[
            pltpu.VMEM((cpw, CHUNK), jnp.int32),
        ]
        + [pltpu.VMEM((CHUNK, D), jnp.float32) for _ in range(RING)]
        + [pltpu.SemaphoreType.DMA for _ in range(2 * RING)],
    )
    def emb(idx_hbm, table_hbm, out_hbm, idx_v, *bufs):
        rows = bufs[:RING]
        gsem = bufs[RING:2 * RING]
        wsem = bufs[2 * RING:3 * RING]
        wid = lax.axis_index("s") * NC + lax.axis_index("c")
        c0 = wid * cpw
        pltpu.sync_copy(idx_hbm.at[pl.ds(c0, cpw)], idx_v)

        for b in range(AHEAD):
            pltpu.async_copy(table_hbm.at[idx_v.at[b]], rows[b], gsem[b])

        def body(i, carry):
            j0 = i * RING
            for b in range(RING):
                j = j0 + b
                jn = j + AHEAD
                bn = (b + AHEAD) % RING

                @pl.when(jnp.logical_and(jn < cpw, jn >= RING))
                def _drain_write():
                    pltpu.make_async_copy(
                        rows[bn], out_hbm.at[pl.ds(0, CHUNK)], wsem[bn]
                    ).wait()

                @pl.when(jn < cpw)
                def _fire_gather():
                    pltpu.async_copy(
                        table_hbm.at[idx_v.at[jn]], rows[bn], gsem[bn]
                    )

                pltpu.make_async_copy(
                    table_hbm.at[idx_v.at[j]], rows[b], gsem[b]
                ).wait()
                pltpu.async_copy(
                    rows[b], out_hbm.at[pl.ds((c0 + j) * CHUNK, CHUNK)], wsem[b]
                )
            return carry

        lax.fori_loop(0, cpw // RING, body, 0)

        for b in range(RING):
            pltpu.make_async_copy(
                rows[b], out_hbm.at[pl.ds(0, CHUNK)], wsem[b]
            ).wait()

    lin = emb(idx2d, tlin)
    ot = _chunk_transpose(lin, F, BT, D)
    o5 = ot.reshape(F, D // 8, BT, 8, CHUNK)
    return o5.transpose((2, 4, 0, 1, 3)).reshape(B, F, D)


# speedup vs baseline: 1.7663x; 1.3198x over previous
import functools

import jax
import jax.numpy as jnp
from jax import lax
from jax.experimental import pallas as pl
from jax.experimental.pallas import tpu as pltpu
from jax.experimental.pallas import tpu_sc as plsc

CHUNK = 128
RING = 8
AHEAD = 4
BR = 2048
TB = 32


def _pack_kernel(p_ref, q_ref, o_ref):
    o_ref[...] = jnp.concatenate([p_ref[...].T, q_ref[...].T], axis=1)


def _pack_table(table):
    V, D = table.shape
    H = V // 2
    tT = table.T
    grid = (H + BR - 1) // BR
    return pl.pallas_call(
        _pack_kernel,
        grid=(grid,),
        in_specs=[
            pl.BlockSpec((D, BR), lambda i: (0, 2 * i)),
            pl.BlockSpec(
                (D, BR), lambda i: (0, jnp.minimum(2 * i + 1, V // BR))
            ),
        ],
        out_specs=pl.BlockSpec((BR, 2 * D), lambda i: (i, 0)),
        out_shape=jax.ShapeDtypeStruct((grid * BR, 2 * D), jnp.float32),
    )(tT, tT)


def _chunk_transpose_kernel(i_ref, o_ref):
    BT = i_ref.shape[0] // 64
    D = i_ref.shape[1] // 2
    for bt in range(BT):
        prT = i_ref[pl.ds(bt * 64, 64)].T
        t = jnp.concatenate([prT[:D], prT[D:]], axis=1)
        for dt in range(D // 8):
            o_ref[dt * BT + bt] = t[dt * 8:(dt + 1) * 8]


def _chunk_transpose(lin128, F, BT, D):
    DT = D // 8
    return pl.pallas_call(
        _chunk_transpose_kernel,
        grid=(F,),
        in_specs=[pl.BlockSpec((BT * 64, 2 * D), lambda i: (i, 0))],
        out_specs=pl.BlockSpec((DT * BT, 8, CHUNK), lambda i: (i, 0, 0)),
        out_shape=jax.ShapeDtypeStruct((F * DT * BT, 8, CHUNK), jnp.float32),
    )(lin128)


def kernel(x, table):
    B, F = x.shape
    V, D = table.shape
    N = B * F
    BT = B // CHUNK
    n_chunks = F * BT
    assert N == n_chunks * CHUNK

    info = plsc.get_sparse_core_info()
    NC, NS = info.num_cores, info.num_subcores
    NW = NC * NS
    assert n_chunks % NW == 0
    cpw = n_chunks // NW
    assert cpw % RING == 0

    xi = x.astype(jnp.int32)
    u = xi % (2 * BR)
    base = xi - u
    v2 = jnp.where(u < BR, 2 * u, 2 * (u - BR) + 1)
    idx2d = (base + v2).T.reshape(n_chunks, 2, CHUNK // 2)
    idx2d = idx2d.transpose(0, 2, 1).reshape(n_chunks, CHUNK)
    tlin = _pack_table(table)
    tlin = tlin.reshape(tlin.shape[0] * 2, D)
    mesh = plsc.VectorSubcoreMesh(core_axis_name="c", subcore_axis_name="s")

    @functools.partial(
        pl.kernel,
        mesh=mesh,
        compiler_params=pltpu.CompilerParams(use_tc_tiling_on_sc=False),
        out_type=jax.ShapeDtypeStruct((N, D), jnp.float32),
        scratch_types=[
            pltpu.VMEM((cpw, CHUNK), jnp.int32),
        ]
        + [pltpu.VMEM((CHUNK, D), jnp.float32) for _ in range(RING)]
        + [pltpu.SemaphoreType.DMA for _ in range(2 * RING)],
    )
    def emb(idx_hbm, table_hbm, out_hbm, idx_v, *bufs):
        rows = bufs[:RING]
        gsem = bufs[RING:2 * RING]
        wsem = bufs[2 * RING:3 * RING]
        wid = lax.axis_index("s") * NC + lax.axis_index("c")
        c0 = wid * cpw
        pltpu.sync_copy(idx_hbm.at[pl.ds(c0, cpw)], idx_v)

        for b in range(AHEAD):
            pltpu.async_copy(table_hbm.at[idx_v.at[b]], rows[b], gsem[b])

        def body(i, carry):
            j0 = i * RING
            for b in range(RING):
                j = j0 + b
                jn = j + AHEAD
                bn = (b + AHEAD) % RING

                @pl.when(jnp.logical_and(jn < cpw, jn >= RING))
                def _drain_write():
                    pltpu.make_async_copy(
                        rows[bn], out_hbm.at[pl.ds(0, CHUNK)], wsem[bn]
                    ).wait()

                @pl.when(jn < cpw)
                def _fire_gather():
                    pltpu.async_copy(
                        table_hbm.at[idx_v.at[jn]], rows[bn], gsem[bn]
                    )

                pltpu.make_async_copy(
                    table_hbm.at[idx_v.at[j]], rows[b], gsem[b]
                ).wait()
                pltpu.async_copy(
                    rows[b], out_hbm.at[pl.ds((c0 + j) * CHUNK, CHUNK)], wsem[b]
                )
            return carry

        lax.fori_loop(0, cpw // RING, body, 0)

        for b in range(RING):
            pltpu.make_async_copy(
                rows[b], out_hbm.at[pl.ds(0, CHUNK)], wsem[b]
            ).wait()

    lin = emb(idx2d, tlin)
    lin128 = lin.reshape(N // 2, 2 * D)
    ot = _chunk_transpose(lin128, F, BT, D)
    o5 = ot.reshape(F, D // 8, BT, 8, CHUNK)
    return o5.transpose((2, 4, 0, 1, 3)).reshape(B, F, D)


# speedup vs baseline: 2.1829x; 1.2359x over previous
import functools

import jax
import jax.numpy as jnp
from jax import lax
from jax.experimental import pallas as pl
from jax.experimental.pallas import tpu as pltpu
from jax.experimental.pallas import tpu_sc as plsc

CHUNK = 128
RING = 8
AHEAD = 4
BR = 8192
TB = 32


def _pack_kernel(p_ref, q_ref, o_ref):
    o_ref[...] = jnp.concatenate([p_ref[...].T, q_ref[...].T], axis=1)


def _pack_table(table):
    V, D = table.shape
    H = V // 2
    tT = table.T
    grid = (H + BR - 1) // BR
    return pl.pallas_call(
        _pack_kernel,
        grid=(grid,),
        in_specs=[
            pl.BlockSpec((D, BR), lambda i: (0, 2 * i)),
            pl.BlockSpec(
                (D, BR), lambda i: (0, jnp.minimum(2 * i + 1, V // BR))
            ),
        ],
        out_specs=pl.BlockSpec((BR, 2 * D), lambda i: (i, 0)),
        out_shape=jax.ShapeDtypeStruct((grid * BR, 2 * D), jnp.float32),
    )(tT, tT)


def _chunk_transpose_kernel(i_ref, o_ref):
    BT = i_ref.shape[0] // 64
    D = i_ref.shape[1] // 2
    for bt in range(BT):
        prT = i_ref[pl.ds(bt * 64, 64)].T
        t = jnp.concatenate([prT[:D], prT[D:]], axis=1)
        for dt in range(D // 8):
            o_ref[dt * BT + bt] = t[dt * 8:(dt + 1) * 8]


def _chunk_transpose(lin128, F, BT, D):
    DT = D // 8
    return pl.pallas_call(
        _chunk_transpose_kernel,
        grid=(F,),
        in_specs=[pl.BlockSpec((BT * 64, 2 * D), lambda i: (i, 0))],
        out_specs=pl.BlockSpec((DT * BT, 8, CHUNK), lambda i: (i, 0, 0)),
        out_shape=jax.ShapeDtypeStruct((F * DT * BT, 8, CHUNK), jnp.float32),
    )(lin128)


def kernel(x, table):
    B, F = x.shape
    V, D = table.shape
    N = B * F
    BT = B // CHUNK
    n_chunks = F * BT
    assert N == n_chunks * CHUNK

    info = plsc.get_sparse_core_info()
    NC, NS = info.num_cores, info.num_subcores
    NW = NC * NS
    assert n_chunks % NW == 0
    cpw = n_chunks // NW
    assert cpw % RING == 0

    xi = x.astype(jnp.int32)
    u = xi % (2 * BR)
    base = xi - u
    v2 = jnp.where(u < BR, 2 * u, 2 * (u - BR) + 1)
    idx2d = (base + v2).T.reshape(n_chunks, 2, CHUNK // 2)
    idx2d = idx2d.transpose(0, 2, 1).reshape(n_chunks, CHUNK)
    tlin = _pack_table(table)
    tlin = tlin.reshape(tlin.shape[0] * 2, D)
    mesh = plsc.VectorSubcoreMesh(core_axis_name="c", subcore_axis_name="s")

    @functools.partial(
        pl.kernel,
        mesh=mesh,
        compiler_params=pltpu.CompilerParams(use_tc_tiling_on_sc=False),
        out_type=jax.ShapeDtypeStruct((N, D), jnp.float32),
        scratch_types=[
            pltpu.VMEM((cpw, CHUNK), jnp.int32),
        ]
        + [pltpu.VMEM((CHUNK, D), jnp.float32) for _ in range(RING)]
        + [pltpu.SemaphoreType.DMA for _ in range(2 * RING)],
    )
    def emb(idx_hbm, table_hbm, out_hbm, idx_v, *bufs):
        rows = bufs[:RING]
        gsem = bufs[RING:2 * RING]
        wsem = bufs[2 * RING:3 * RING]
        wid = lax.axis_index("s") * NC + lax.axis_index("c")
        c0 = wid * cpw
        pltpu.sync_copy(idx_hbm.at[pl.ds(c0, cpw)], idx_v)

        for b in range(AHEAD):
            pltpu.async_copy(table_hbm.at[idx_v.at[b]], rows[b], gsem[b])

        def body(i, carry):
            j0 = i * RING
            for b in range(RING):
                j = j0 + b
                jn = j + AHEAD
                bn = (b + AHEAD) % RING

                @pl.when(jnp.logical_and(jn < cpw, jn >= RING))
                def _drain_write():
                    pltpu.make_async_copy(
                        rows[bn], out_hbm.at[pl.ds(0, CHUNK)], wsem[bn]
                    ).wait()

                @pl.when(jn < cpw)
                def _fire_gather():
                    pltpu.async_copy(
                        table_hbm.at[idx_v.at[jn]], rows[bn], gsem[bn]
                    )

                pltpu.make_async_copy(
                    table_hbm.at[idx_v.at[j]], rows[b], gsem[b]
                ).wait()
                pltpu.async_copy(
                    rows[b], out_hbm.at[pl.ds((c0 + j) * CHUNK, CHUNK)], wsem[b]
                )
            return carry

        lax.fori_loop(0, cpw // RING, body, 0)

        for b in range(RING):
            pltpu.make_async_copy(
                rows[b], out_hbm.at[pl.ds(0, CHUNK)], wsem[b]
            ).wait()

    lin = emb(idx2d, tlin)
    lin128 = lin.reshape(N // 2, 2 * D)
    ot = _chunk_transpose(lin128, F, BT, D)
    o5 = ot.reshape(F, D // 8, BT, 8, CHUNK)
    return o5.transpose((2, 4, 0, 1, 3)).reshape(B, F, D)


# speedup vs baseline: 2.2523x; 1.0318x over previous
import functools

import jax
import jax.numpy as jnp
from jax import lax
from jax.experimental import pallas as pl
from jax.experimental.pallas import tpu as pltpu
from jax.experimental.pallas import tpu_sc as plsc

CHUNK = 128
RING = 8
AHEAD = 4
BR = 16384
TB = 32


def _pack_kernel(p_ref, q_ref, o_ref):
    o_ref[...] = jnp.concatenate([p_ref[...].T, q_ref[...].T], axis=1)


def _pack_table(table):
    V, D = table.shape
    H = V // 2
    tT = table.T
    grid = (H + BR - 1) // BR
    return pl.pallas_call(
        _pack_kernel,
        grid=(grid,),
        in_specs=[
            pl.BlockSpec((D, BR), lambda i: (0, 2 * i)),
            pl.BlockSpec(
                (D, BR), lambda i: (0, jnp.minimum(2 * i + 1, V // BR))
            ),
        ],
        out_specs=pl.BlockSpec((BR, 2 * D), lambda i: (i, 0)),
        out_shape=jax.ShapeDtypeStruct((grid * BR, 2 * D), jnp.float32),
    )(tT, tT)


def _chunk_transpose_kernel(i_ref, o_ref):
    BT = i_ref.shape[0] // 64
    D = i_ref.shape[1] // 2
    for bt in range(BT):
        prT = i_ref[pl.ds(bt * 64, 64)].T
        t = jnp.concatenate([prT[:D], prT[D:]], axis=1)
        for dt in range(D // 8):
            o_ref[dt * BT + bt] = t[dt * 8:(dt + 1) * 8]


def _chunk_transpose(lin128, F, BT, D):
    DT = D // 8
    return pl.pallas_call(
        _chunk_transpose_kernel,
        grid=(F,),
        in_specs=[pl.BlockSpec((BT * 64, 2 * D), lambda i: (i, 0))],
        out_specs=pl.BlockSpec((DT * BT, 8, CHUNK), lambda i: (i, 0, 0)),
        out_shape=jax.ShapeDtypeStruct((F * DT * BT, 8, CHUNK), jnp.float32),
    )(lin128)


def kernel(x, table):
    B, F = x.shape
    V, D = table.shape
    N = B * F
    BT = B // CHUNK
    n_chunks = F * BT
    assert N == n_chunks * CHUNK

    info = plsc.get_sparse_core_info()
    NC, NS = info.num_cores, info.num_subcores
    NW = NC * NS
    assert n_chunks % NW == 0
    cpw = n_chunks // NW
    assert cpw % RING == 0

    xi = x.astype(jnp.int32)
    u = xi % (2 * BR)
    base = xi - u
    v2 = jnp.where(u < BR, 2 * u, 2 * (u - BR) + 1)
    idx2d = (base + v2).T.reshape(n_chunks, 2, CHUNK // 2)
    idx2d = idx2d.transpose(0, 2, 1).reshape(n_chunks, CHUNK)
    tlin = _pack_table(table)
    tlin = tlin.reshape(tlin.shape[0] * 2, D)
    mesh = plsc.VectorSubcoreMesh(core_axis_name="c", subcore_axis_name="s")

    @functools.partial(
        pl.kernel,
        mesh=mesh,
        compiler_params=pltpu.CompilerParams(use_tc_tiling_on_sc=False),
        out_type=jax.ShapeDtypeStruct((N, D), jnp.float32),
        scratch_types=[
            pltpu.VMEM((cpw, CHUNK), jnp.int32),
        ]
        + [pltpu.VMEM((CHUNK, D), jnp.float32) for _ in range(RING)]
        + [pltpu.SemaphoreType.DMA for _ in range(2 * RING)],
    )
    def emb(idx_hbm, table_hbm, out_hbm, idx_v, *bufs):
        rows = bufs[:RING]
        gsem = bufs[RING:2 * RING]
        wsem = bufs[2 * RING:3 * RING]
        wid = lax.axis_index("s") * NC + lax.axis_index("c")
        c0 = wid * cpw
        pltpu.sync_copy(idx_hbm.at[pl.ds(c0, cpw)], idx_v)

        for b in range(AHEAD):
            pltpu.async_copy(table_hbm.at[idx_v.at[b]], rows[b], gsem[b])

        def body(i, carry):
            j0 = i * RING
            for b in range(RING):
                j = j0 + b
                jn = j + AHEAD
                bn = (b + AHEAD) % RING

                @pl.when(jnp.logical_and(jn < cpw, jn >= RING))
                def _drain_write():
                    pltpu.make_async_copy(
                        rows[bn], out_hbm.at[pl.ds(0, CHUNK)], wsem[bn]
                    ).wait()

                @pl.when(jn < cpw)
                def _fire_gather():
                    pltpu.async_copy(
                        table_hbm.at[idx_v.at[jn]], rows[bn], gsem[bn]
                    )

                pltpu.make_async_copy(
                    table_hbm.at[idx_v.at[j]], rows[b], gsem[b]
                ).wait()
                pltpu.async_copy(
                    rows[b], out_hbm.at[pl.ds((c0 + j) * CHUNK, CHUNK)], wsem[b]
                )
            return carry

        lax.fori_loop(0, cpw // RING, body, 0)

        for b in range(RING):
            pltpu.make_async_copy(
                rows[b], out_hbm.at[pl.ds(0, CHUNK)], wsem[b]
            ).wait()

    lin = emb(idx2d, tlin)
    lin128 = lin.reshape(N // 2, 2 * D)
    ot = _chunk_transpose(lin128, F, BT, D)
    o5 = ot.reshape(F, D // 8, BT, 8, CHUNK)
    return o5.transpose((2, 4, 0, 1, 3)).reshape(B, F, D)
